# Initial kernel scaffold; baseline (speedup 1.0000x reference)
import jax
import jax.numpy as jnp
from jax.experimental import pallas as pl


def kernel(x, edge_index, W1, b1, W2, b2):
    raise NotImplementedError("write your pallas kernel here")


# speedup vs baseline: 1.5151x; 1.5151x over previous
import functools

import jax
import jax.numpy as jnp
from jax import lax
from jax.experimental import pallas as pl
from jax.experimental.pallas import tpu as pltpu
from jax.experimental.pallas import tpu_sc as plsc

N = 10000
D = 128
DH = D // 2
NC, NS = 2, 16
CHUNK = 128
E = 320000
K = -(-E // (NS * CHUNK))
EPAD = NS * K * CHUNK
ACC_ROWS = ((N + 1 + 127) // 128) * 128
RPT = ACC_ROWS // NS


def _sc_scatter(table, src_slab, dst_slab, zrows):
    mesh = plsc.VectorSubcoreMesh(core_axis_name="c", subcore_axis_name="s")

    @functools.partial(
        pl.kernel,
        out_type=jax.ShapeDtypeStruct((NC, ACC_ROWS, DH), jnp.float32),
        mesh=mesh,
        scratch_types=[
            pltpu.VMEM((K, CHUNK), jnp.int32),
            pltpu.VMEM((K, CHUNK), jnp.int32),
            pltpu.VMEM((CHUNK, DH), jnp.float32),
            pltpu.VMEM((CHUNK, DH), jnp.float32),
            pltpu.VMEM_SHARED((ACC_ROWS, DH), jnp.float32),
            pltpu.SemaphoreType.DMA,
            pltpu.SemaphoreType.DMA,
        ],
        compiler_params=pltpu.CompilerParams(use_tc_tiling_on_sc=False),
    )
    def k(table_hbm, src_hbm, dst_hbm, z_hbm, out_hbm,
          src_v, dst_v, buf0, buf1, acc, sem0, sem1):
        c = lax.axis_index("c")
        s = lax.axis_index("s")
        pltpu.sync_copy(z_hbm, acc.at[pl.ds(s * RPT, RPT)])
        pltpu.sync_copy(src_hbm.at[c, s], src_v)
        pltpu.sync_copy(dst_hbm.at[s], dst_v)
        plsc.subcore_barrier()

        pltpu.async_copy(table_hbm.at[src_v.at[0]], buf0, sem0)

        @pl.loop(0, K - 1, step=2)
        def _(j):
            pltpu.async_copy(table_hbm.at[src_v.at[j + 1]], buf1, sem1)
            pltpu.make_async_copy(table_hbm.at[src_v.at[j]], buf0, sem0).wait()
            pltpu.sync_copy(buf0, acc.at[dst_v.at[j]], add=True)
            pltpu.async_copy(table_hbm.at[src_v.at[j + 2]], buf0, sem0)
            pltpu.make_async_copy(table_hbm.at[src_v.at[j + 1]], buf1, sem1).wait()
            pltpu.sync_copy(buf1, acc.at[dst_v.at[j + 1]], add=True)

        pltpu.make_async_copy(table_hbm.at[src_v.at[K - 1]], buf0, sem0).wait()
        pltpu.sync_copy(buf0, acc.at[dst_v.at[K - 1]], add=True)
        plsc.subcore_barrier()
        pltpu.sync_copy(acc.at[pl.ds(s * RPT, RPT)],
                        out_hbm.at[c, pl.ds(s * RPT, RPT)])

    return k(table, src_slab, dst_slab, zrows)


def _tc1(x, W, d0):
    BR = 1000

    def body(x_ref, w_ref, d0_ref, y_ref, dis_ref):
        deg = d0_ref[...] + 1.0
        dis = lax.rsqrt(deg)
        dis_ref[...] = dis
        y_ref[...] = dis * jnp.dot(x_ref[...], w_ref[...],
                                   preferred_element_type=jnp.float32)

    return pl.pallas_call(
        body,
        grid=(N // BR,),
        in_specs=[
            pl.BlockSpec((BR, D), lambda j: (j, 0)),
            pl.BlockSpec((D, D), lambda j: (0, 0)),
            pl.BlockSpec((BR, D), lambda j: (j, 0)),
        ],
        out_specs=[
            pl.BlockSpec((BR, D), lambda j: (j, 0)),
            pl.BlockSpec((BR, D), lambda j: (j, 0)),
        ],
        out_shape=[
            jax.ShapeDtypeStruct((N, D), jnp.float32),
            jax.ShapeDtypeStruct((N, D), jnp.float32),
        ],
    )(x, W, d0)


def _tc2(a0, a1, y1, dis, b, W):
    BR = 1000

    def body(a0_ref, a1_ref, y1_ref, dis_ref, b_ref, w_ref, y2_ref):
        dis = dis_ref[...]
        a = jnp.concatenate([a0_ref[...], a1_ref[...]], axis=1)
        h = jnp.tanh(dis * (a + y1_ref[...]) + b_ref[...])
        y2_ref[...] = dis * jnp.dot(h, w_ref[...],
                                    preferred_element_type=jnp.float32)

    return pl.pallas_call(
        body,
        grid=(N // BR,),
        in_specs=[
            pl.BlockSpec((BR, DH), lambda j: (j, 0)),
            pl.BlockSpec((BR, DH), lambda j: (j, 0)),
            pl.BlockSpec((BR, D), lambda j: (j, 0)),
            pl.BlockSpec((BR, D), lambda j: (j, 0)),
            pl.BlockSpec((1, D), lambda j: (0, 0)),
            pl.BlockSpec((D, D), lambda j: (0, 0)),
        ],
        out_specs=pl.BlockSpec((BR, D), lambda j: (j, 0)),
        out_shape=jax.ShapeDtypeStruct((N, D), jnp.float32),
    )(a0, a1, y1, dis, b, W)


def _tc3(a0, a1, y2, dis, b):
    BR = 1000

    def body(a0_ref, a1_ref, y2_ref, dis_ref, b_ref, o_ref):
        a = jnp.concatenate([a0_ref[...], a1_ref[...]], axis=1)
        o_ref[...] = jnp.tanh(
            dis_ref[...] * (a + y2_ref[...]) + b_ref[...])

    return pl.pallas_call(
        body,
        grid=(N // BR,),
        in_specs=[
            pl.BlockSpec((BR, DH), lambda j: (j, 0)),
            pl.BlockSpec((BR, DH), lambda j: (j, 0)),
            pl.BlockSpec((BR, D), lambda j: (j, 0)),
            pl.BlockSpec((BR, D), lambda j: (j, 0)),
            pl.BlockSpec((1, D), lambda j: (0, 0)),
        ],
        out_specs=pl.BlockSpec((BR, D), lambda j: (j, 0)),
        out_shape=jax.ShapeDtypeStruct((N, D), jnp.float32),
    )(a0, a1, y2, dis, b)


def _split_cols(y):
    return jnp.concatenate([y[:, :DH], y[:, DH:]], axis=0)


def kernel(x, edge_index, W1, b1, W2, b2):
    src = edge_index[0]
    dst = edge_index[1]
    pad = EPAD - E
    src_base = jnp.concatenate(
        [src, jnp.zeros((pad,), jnp.int32)]).reshape(NS, K, CHUNK)
    srcp = src_base[None] + (jnp.arange(NC, dtype=jnp.int32) * N)[:, None, None, None]
    dstp = jnp.concatenate(
        [dst, jnp.full((pad,), N, jnp.int32)]).reshape(NS, K, CHUNK)
    zrows = jnp.zeros((RPT, DH), jnp.float32)

    ones_t = jnp.ones((2 * 8, DH), jnp.float32)
    src_deg = jnp.broadcast_to(
        (jnp.arange(NC, dtype=jnp.int32) * 8)[:, None, None, None],
        (NC, NS, K, CHUNK))
    accd = _sc_scatter(ones_t, src_deg, dstp, zrows)
    deg0 = jnp.concatenate([accd[0, :N, :], accd[0, :N, :]], axis=1)

    y1, dis = _tc1(x, W1, deg0)
    acc1 = _sc_scatter(_split_cols(y1), srcp, dstp, zrows)
    y2 = _tc2(acc1[0, :N, :], acc1[1, :N, :], y1, dis, b1.reshape(1, D), W2)
    acc2 = _sc_scatter(_split_cols(y2), srcp, dstp, zrows)
    return _tc3(acc2[0, :N, :], acc2[1, :N, :], y2, dis, b2.reshape(1, D))


# speedup vs baseline: 20.4622x; 13.5055x over previous
import functools

import jax
import jax.numpy as jnp
from jax import lax
from jax.experimental import pallas as pl
from jax.experimental.pallas import tpu as pltpu
from jax.experimental.pallas import tpu_sc as plsc

N = 10000
D = 128
DH = D // 2
NC, NS = 2, 16
CHUNK = 128
E = 320000
K = -(-E // (NS * CHUNK))
EPAD = NS * K * CHUNK
ACC_ROWS = ((N + 1 + 127) // 128) * 128
RPT = ACC_ROWS // NS


def _sc_scatter(table, src_slab, dst_slab, zrows):
    mesh = plsc.VectorSubcoreMesh(core_axis_name="c", subcore_axis_name="s")

    @functools.partial(
        pl.kernel,
        out_type=jax.ShapeDtypeStruct((NC, ACC_ROWS, DH), jnp.float32),
        mesh=mesh,
        scratch_types=[
            pltpu.VMEM((K, CHUNK), jnp.int32),
            pltpu.VMEM((K, CHUNK), jnp.int32),
            pltpu.VMEM((CHUNK, DH), jnp.float32),
            pltpu.VMEM((CHUNK, DH), jnp.float32),
            pltpu.VMEM_SHARED((ACC_ROWS, DH), jnp.float32),
            pltpu.SemaphoreType.DMA,
            pltpu.SemaphoreType.DMA,
        ],
        compiler_params=pltpu.CompilerParams(use_tc_tiling_on_sc=False),
    )
    def k(table_hbm, src_hbm, dst_hbm, z_hbm, out_hbm,
          src_v, dst_v, buf0, buf1, acc, sem0, sem1):
        c = lax.axis_index("c")
        s = lax.axis_index("s")
        pltpu.sync_copy(z_hbm, acc.at[pl.ds(s * RPT, RPT)])
        pltpu.sync_copy(src_hbm.at[c, s], src_v)
        pltpu.sync_copy(dst_hbm.at[s], dst_v)
        plsc.subcore_barrier()

        pltpu.async_copy(table_hbm.at[src_v.at[0]], buf0, sem0)

        @pl.loop(0, K - 1, step=2)
        def _(j):
            pltpu.async_copy(table_hbm.at[src_v.at[j + 1]], buf1, sem1)
            pltpu.make_async_copy(table_hbm.at[src_v.at[j]], buf0, sem0).wait()
            pltpu.sync_copy(buf0, acc.at[dst_v.at[j]], add=True)
            pltpu.async_copy(table_hbm.at[src_v.at[j + 2]], buf0, sem0)
            pltpu.make_async_copy(table_hbm.at[src_v.at[j + 1]], buf1, sem1).wait()
            pltpu.sync_copy(buf1, acc.at[dst_v.at[j + 1]], add=True)

        pltpu.make_async_copy(table_hbm.at[src_v.at[K - 1]], buf0, sem0).wait()
        pltpu.sync_copy(buf0, acc.at[dst_v.at[K - 1]], add=True)
        plsc.subcore_barrier()
        pltpu.sync_copy(acc.at[pl.ds(s * RPT, RPT)],
                        out_hbm.at[c, pl.ds(s * RPT, RPT)])

    return k(table, src_slab, dst_slab, zrows)


DEGW = 16


def _sc_degree(dst_slab, zdeg):
    mesh = plsc.VectorSubcoreMesh(core_axis_name="c", subcore_axis_name="s")

    @functools.partial(
        pl.kernel,
        out_type=jax.ShapeDtypeStruct((NC, ACC_ROWS, DEGW), jnp.float32),
        mesh=mesh,
        scratch_types=[
            pltpu.VMEM((K, CHUNK), jnp.int32),
            pltpu.VMEM((CHUNK, DEGW), jnp.float32),
            pltpu.VMEM_SHARED((ACC_ROWS, DEGW), jnp.float32),
            pltpu.SemaphoreType.DMA,
        ],
        compiler_params=pltpu.CompilerParams(use_tc_tiling_on_sc=False),
    )
    def k(dst_hbm, z_hbm, out_hbm, dst_v, ones_v, acc, sem):
        c = lax.axis_index("c")
        s = lax.axis_index("s")
        pltpu.sync_copy(z_hbm, acc.at[pl.ds(s * RPT, RPT)])
        pltpu.sync_copy(dst_hbm.at[s], dst_v)
        for i in range(CHUNK):
            ones_v[i] = jnp.ones((DEGW,), jnp.float32)
        plsc.subcore_barrier()

        @pl.loop(0, K)
        def _(j):
            pltpu.async_copy(ones_v, acc.at[dst_v.at[j]], sem, add=True)

        @pl.loop(0, K)
        def _(j):
            pltpu.make_async_copy(ones_v, acc.at[dst_v.at[j]], sem).wait()

        plsc.subcore_barrier()
        pltpu.sync_copy(acc.at[pl.ds(s * RPT, RPT)],
                        out_hbm.at[c, pl.ds(s * RPT, RPT)])

    return k(dst_slab, zdeg)


def _tc1(x, W, d0):
    BR = 1000

    def body(x_ref, w_ref, d0_ref, y_ref, dis_ref):
        deg = d0_ref[...][:, 0:1] + 1.0
        dis = jnp.broadcast_to(lax.rsqrt(deg), (BR, D))
        dis_ref[...] = dis
        y_ref[...] = dis * jnp.dot(x_ref[...], w_ref[...],
                                   preferred_element_type=jnp.float32)

    return pl.pallas_call(
        body,
        grid=(N // BR,),
        in_specs=[
            pl.BlockSpec((BR, D), lambda j: (j, 0)),
            pl.BlockSpec((D, D), lambda j: (0, 0)),
            pl.BlockSpec((BR, DEGW), lambda j: (j, 0)),
        ],
        out_specs=[
            pl.BlockSpec((BR, D), lambda j: (j, 0)),
            pl.BlockSpec((BR, D), lambda j: (j, 0)),
        ],
        out_shape=[
            jax.ShapeDtypeStruct((N, D), jnp.float32),
            jax.ShapeDtypeStruct((N, D), jnp.float32),
        ],
    )(x, W, d0)


def _tc2(a0, a1, y1, dis, b, W):
    BR = 1000

    def body(a0_ref, a1_ref, y1_ref, dis_ref, b_ref, w_ref, y2_ref):
        dis = dis_ref[...]
        a = jnp.concatenate([a0_ref[...], a1_ref[...]], axis=1)
        h = jnp.tanh(dis * (a + y1_ref[...]) + b_ref[...])
        y2_ref[...] = dis * jnp.dot(h, w_ref[...],
                                    preferred_element_type=jnp.float32)

    return pl.pallas_call(
        body,
        grid=(N // BR,),
        in_specs=[
            pl.BlockSpec((BR, DH), lambda j: (j, 0)),
            pl.BlockSpec((BR, DH), lambda j: (j, 0)),
            pl.BlockSpec((BR, D), lambda j: (j, 0)),
            pl.BlockSpec((BR, D), lambda j: (j, 0)),
            pl.BlockSpec((1, D), lambda j: (0, 0)),
            pl.BlockSpec((D, D), lambda j: (0, 0)),
        ],
        out_specs=pl.BlockSpec((BR, D), lambda j: (j, 0)),
        out_shape=jax.ShapeDtypeStruct((N, D), jnp.float32),
    )(a0, a1, y1, dis, b, W)


def _tc3(a0, a1, y2, dis, b):
    BR = 1000

    def body(a0_ref, a1_ref, y2_ref, dis_ref, b_ref, o_ref):
        a = jnp.concatenate([a0_ref[...], a1_ref[...]], axis=1)
        o_ref[...] = jnp.tanh(
            dis_ref[...] * (a + y2_ref[...]) + b_ref[...])

    return pl.pallas_call(
        body,
        grid=(N // BR,),
        in_specs=[
            pl.BlockSpec((BR, DH), lambda j: (j, 0)),
            pl.BlockSpec((BR, DH), lambda j: (j, 0)),
            pl.BlockSpec((BR, D), lambda j: (j, 0)),
            pl.BlockSpec((BR, D), lambda j: (j, 0)),
            pl.BlockSpec((1, D), lambda j: (0, 0)),
        ],
        out_specs=pl.BlockSpec((BR, D), lambda j: (j, 0)),
        out_shape=jax.ShapeDtypeStruct((N, D), jnp.float32),
    )(a0, a1, y2, dis, b)


def _split_cols(y):
    return jnp.concatenate([y[:, :DH], y[:, DH:]], axis=0)


def kernel(x, edge_index, W1, b1, W2, b2):
    src = edge_index[0]
    dst = edge_index[1]
    pad = EPAD - E
    src_base = jnp.concatenate(
        [src, jnp.zeros((pad,), jnp.int32)]).reshape(NS, K, CHUNK)
    srcp = src_base[None] + (jnp.arange(NC, dtype=jnp.int32) * N)[:, None, None, None]
    dstp = jnp.concatenate(
        [dst, jnp.full((pad,), N, jnp.int32)]).reshape(NS, K, CHUNK)
    zrows = jnp.zeros((RPT, DH), jnp.float32)

    accd = _sc_degree(dstp, jnp.zeros((RPT, DEGW), jnp.float32))

    y1, dis = _tc1(x, W1, accd[0, :N, :])
    acc1 = _sc_scatter(_split_cols(y1), srcp, dstp, zrows)
    y2 = _tc2(acc1[0, :N, :], acc1[1, :N, :], y1, dis, b1.reshape(1, D), W2)
    acc2 = _sc_scatter(_split_cols(y2), srcp, dstp, zrows)
    return _tc3(acc2[0, :N, :], acc2[1, :N, :], y2, dis, b2.reshape(1, D))


# speedup vs baseline: 23.1847x; 1.1330x over previous
import functools

import jax
import jax.numpy as jnp
from jax import lax
from jax.experimental import pallas as pl
from jax.experimental.pallas import tpu as pltpu
from jax.experimental.pallas import tpu_sc as plsc

N = 10000
D = 128
DH = D // 2
NC, NS = 2, 16
LANES = 16
CHUNK = 128
E = 320000
K = -(-E // (NS * CHUNK))
EPAD = NS * K * CHUNK
ACC_ROWS = ((N + 1 + 127) // 128) * 128
RPT = ACC_ROWS // NS
NBUF = 4
KFULL = (K // NBUF) * NBUF
DEGW = 16


def _sc_scatter(table, src_slab, dst_slab, zrows):
    mesh = plsc.VectorSubcoreMesh(core_axis_name="c", subcore_axis_name="s")

    @functools.partial(
        pl.kernel,
        out_type=jax.ShapeDtypeStruct((NC, ACC_ROWS, DH), jnp.float32),
        mesh=mesh,
        scratch_types=[
            pltpu.VMEM((K, CHUNK), jnp.int32),
            pltpu.VMEM((K, CHUNK), jnp.int32),
            [pltpu.VMEM((CHUNK, DH), jnp.float32) for _ in range(NBUF)],
            pltpu.VMEM_SHARED((ACC_ROWS, DH), jnp.float32),
            [pltpu.SemaphoreType.DMA for _ in range(NBUF)],
            [pltpu.SemaphoreType.DMA for _ in range(NBUF)],
        ],
        compiler_params=pltpu.CompilerParams(use_tc_tiling_on_sc=False),
    )
    def k(table_hbm, src_hbm, dst_hbm, z_hbm, out_hbm,
          src_v, dst_v, bufs, acc, gsems, ssems):
        c = lax.axis_index("c")
        s = lax.axis_index("s")
        pltpu.sync_copy(z_hbm, acc.at[pl.ds(s * RPT, RPT)])
        pltpu.sync_copy(src_hbm.at[s], src_v)
        pltpu.sync_copy(dst_hbm.at[s], dst_v)
        off = c * N

        @pl.loop(0, K)
        def _(r):
            for i in range(CHUNK // LANES):
                sl = pl.ds(i * LANES, LANES)
                src_v[r, sl] = src_v[r, sl] + off

        plsc.subcore_barrier()

        def gather(t, b):
            pltpu.async_copy(table_hbm.at[src_v.at[t]], bufs[b], gsems[b])

        def gather_wait(t, b):
            pltpu.make_async_copy(
                table_hbm.at[src_v.at[t]], bufs[b], gsems[b]).wait()

        def scat(t, b):
            pltpu.async_copy(bufs[b], acc.at[dst_v.at[t]], ssems[b], add=True)

        def scat_wait(t, b):
            pltpu.make_async_copy(bufs[b], acc.at[dst_v.at[t]], ssems[b]).wait()

        for b in range(NBUF):
            gather(b, b)

        @pl.loop(0, KFULL, step=NBUF)
        def _(j):
            for b in range(NBUF):
                gather_wait(j + b, b)
                scat(j + b, b)
            for b in range(NBUF):
                scat_wait(j + b, b)

                @pl.when(j + NBUF + b < K)
                def _():
                    gather(j + NBUF + b, b)

        for t in range(KFULL, K):
            b = t % NBUF
            gather_wait(t, b)
            pltpu.sync_copy(bufs[b], acc.at[dst_v.at[t]], add=True)

        plsc.subcore_barrier()
        pltpu.sync_copy(acc.at[pl.ds(s * RPT, RPT)],
                        out_hbm.at[c, pl.ds(s * RPT, RPT)])

    return k(table, src_slab, dst_slab, zrows)


def _sc_degree(dst_slab, zdeg):
    mesh = plsc.VectorSubcoreMesh(core_axis_name="c", subcore_axis_name="s")

    @functools.partial(
        pl.kernel,
        out_type=jax.ShapeDtypeStruct((NC, ACC_ROWS, DEGW), jnp.float32),
        mesh=mesh,
        scratch_types=[
            pltpu.VMEM((K, CHUNK), jnp.int32),
            pltpu.VMEM((CHUNK, DEGW), jnp.float32),
            pltpu.VMEM_SHARED((ACC_ROWS, DEGW), jnp.float32),
            pltpu.SemaphoreType.DMA,
        ],
        compiler_params=pltpu.CompilerParams(use_tc_tiling_on_sc=False),
    )
    def k(dst_hbm, z_hbm, out_hbm, dst_v, ones_v, acc, sem):
        c = lax.axis_index("c")
        s = lax.axis_index("s")
        pltpu.sync_copy(z_hbm, acc.at[pl.ds(s * RPT, RPT)])
        pltpu.sync_copy(dst_hbm.at[s], dst_v)
        for i in range(CHUNK):
            ones_v[i] = jnp.ones((DEGW,), jnp.float32)
        plsc.subcore_barrier()

        @pl.loop(0, K)
        def _(j):
            pltpu.async_copy(ones_v, acc.at[dst_v.at[j]], sem, add=True)

        @pl.loop(0, K)
        def _(j):
            pltpu.make_async_copy(ones_v, acc.at[dst_v.at[j]], sem).wait()

        plsc.subcore_barrier()
        pltpu.sync_copy(acc.at[pl.ds(s * RPT, RPT)],
                        out_hbm.at[c, pl.ds(s * RPT, RPT)])

    return k(dst_slab, zdeg)


def _dis_block(deg_ref, br):
    return lax.rsqrt(deg_ref[0][:, 0:1] + 1.0)


def _tc1(x, W, accd):
    BR = 1000

    def body(x_ref, w_ref, deg_ref, y_ref):
        dis = _dis_block(deg_ref, BR)
        y = dis * jnp.dot(x_ref[...], w_ref[...],
                          preferred_element_type=jnp.float32)
        y_ref[0] = y[:, :DH]
        y_ref[1] = y[:, DH:]

    return pl.pallas_call(
        body,
        grid=(N // BR,),
        in_specs=[
            pl.BlockSpec((BR, D), lambda j: (j, 0)),
            pl.BlockSpec((D, D), lambda j: (0, 0)),
            pl.BlockSpec((1, BR, DEGW), lambda j: (0, j, 0)),
        ],
        out_specs=pl.BlockSpec((NC, BR, DH), lambda j: (0, j, 0)),
        out_shape=jax.ShapeDtypeStruct((NC, N, DH), jnp.float32),
    )(x, W, accd)


def _tc2(acc, yt, accd, b, W):
    BR = 1000

    def body(a_ref, y_ref, deg_ref, b_ref, w_ref, o_ref):
        dis = _dis_block(deg_ref, BR)
        s = jnp.concatenate(
            [a_ref[0] + y_ref[0], a_ref[1] + y_ref[1]], axis=1)
        h = jnp.tanh(dis * s + b_ref[...])
        y2 = dis * jnp.dot(h, w_ref[...], preferred_element_type=jnp.float32)
        o_ref[0] = y2[:, :DH]
        o_ref[1] = y2[:, DH:]

    return pl.pallas_call(
        body,
        grid=(N // BR,),
        in_specs=[
            pl.BlockSpec((NC, BR, DH), lambda j: (0, j, 0)),
            pl.BlockSpec((NC, BR, DH), lambda j: (0, j, 0)),
            pl.BlockSpec((1, BR, DEGW), lambda j: (0, j, 0)),
            pl.BlockSpec((1, D), lambda j: (0, 0)),
            pl.BlockSpec((D, D), lambda j: (0, 0)),
        ],
        out_specs=pl.BlockSpec((NC, BR, DH), lambda j: (0, j, 0)),
        out_shape=jax.ShapeDtypeStruct((NC, N, DH), jnp.float32),
    )(acc, yt, accd, b, W)


def _tc3(acc, yt, accd, b):
    BR = 1000

    def body(a_ref, y_ref, deg_ref, b_ref, o_ref):
        dis = _dis_block(deg_ref, BR)
        s = jnp.concatenate(
            [a_ref[0] + y_ref[0], a_ref[1] + y_ref[1]], axis=1)
        o_ref[...] = jnp.tanh(dis * s + b_ref[...])

    return pl.pallas_call(
        body,
        grid=(N // BR,),
        in_specs=[
            pl.BlockSpec((NC, BR, DH), lambda j: (0, j, 0)),
            pl.BlockSpec((NC, BR, DH), lambda j: (0, j, 0)),
            pl.BlockSpec((1, BR, DEGW), lambda j: (0, j, 0)),
            pl.BlockSpec((1, D), lambda j: (0, 0)),
        ],
        out_specs=pl.BlockSpec((BR, D), lambda j: (j, 0)),
        out_shape=jax.ShapeDtypeStruct((N, D), jnp.float32),
    )(acc, yt, accd, b)


def kernel(x, edge_index, W1, b1, W2, b2):
    src = edge_index[0]
    dst = edge_index[1]
    pad = EPAD - E
    srcp = jnp.concatenate(
        [src, jnp.zeros((pad,), jnp.int32)]).reshape(NS, K, CHUNK)
    dstp = jnp.concatenate(
        [dst, jnp.full((pad,), N, jnp.int32)]).reshape(NS, K, CHUNK)
    zrows = jnp.zeros((RPT, DH), jnp.float32)

    accd = _sc_degree(dstp, jnp.zeros((RPT, DEGW), jnp.float32))

    y1t = _tc1(x, W1, accd)
    acc1 = _sc_scatter(y1t.reshape(NC * N, DH), srcp, dstp, zrows)
    y2t = _tc2(acc1, y1t, accd, b1.reshape(1, D), W2)
    acc2 = _sc_scatter(y2t.reshape(NC * N, DH), srcp, dstp, zrows)
    return _tc3(acc2, y2t, accd, b2.reshape(1, D))


# speedup vs baseline: 24.3439x; 1.0500x over previous
import functools

import jax
import jax.numpy as jnp
from jax import lax
from jax.experimental import pallas as pl
from jax.experimental.pallas import tpu as pltpu
from jax.experimental.pallas import tpu_sc as plsc

N = 10000
D = 128
DH = D // 2
NC, NS = 2, 16
LANES = 16
CHUNK = 128
E = 320000
K = -(-E // (NS * CHUNK))
EPAD = NS * K * CHUNK
ACC_ROWS = ((N + 1 + 127) // 128) * 128
RPT = ACC_ROWS // NS
NBUF = 6
KFULL = (K // NBUF) * NBUF
KHALF = -(-K // 2)
DEGW = 16


def _sc_scatter(table, src_slab, dst_slab, zrows):
    mesh = plsc.VectorSubcoreMesh(core_axis_name="c", subcore_axis_name="s")

    @functools.partial(
        pl.kernel,
        out_type=jax.ShapeDtypeStruct((NC, ACC_ROWS, DH), jnp.float32),
        mesh=mesh,
        scratch_types=[
            pltpu.VMEM((K, CHUNK), jnp.int32),
            pltpu.VMEM((K, CHUNK), jnp.int32),
            [pltpu.VMEM((CHUNK, DH), jnp.float32) for _ in range(NBUF)],
            pltpu.VMEM_SHARED((ACC_ROWS, DH), jnp.float32),
            [pltpu.SemaphoreType.DMA for _ in range(NBUF)],
            [pltpu.SemaphoreType.DMA for _ in range(NBUF)],
        ],
        compiler_params=pltpu.CompilerParams(use_tc_tiling_on_sc=False),
    )
    def k(table_hbm, src_hbm, dst_hbm, z_hbm, out_hbm,
          src_v, dst_v, bufs, acc, gsems, ssems):
        c = lax.axis_index("c")
        s = lax.axis_index("s")
        pltpu.sync_copy(z_hbm, acc.at[pl.ds(s * RPT, RPT)])
        pltpu.sync_copy(src_hbm.at[s], src_v)
        pltpu.sync_copy(dst_hbm.at[s], dst_v)
        off = c * N

        @pl.loop(0, K)
        def _(r):
            for i in range(CHUNK // LANES):
                sl = pl.ds(i * LANES, LANES)
                src_v[r, sl] = src_v[r, sl] + off

        plsc.subcore_barrier()

        def gather(t, b):
            pltpu.async_copy(table_hbm.at[src_v.at[t]], bufs[b], gsems[b])

        def gather_wait(t, b):
            pltpu.make_async_copy(
                table_hbm.at[src_v.at[t]], bufs[b], gsems[b]).wait()

        def scat(t, b):
            pltpu.async_copy(bufs[b], acc.at[dst_v.at[t]], ssems[b], add=True)

        def scat_wait(t, b):
            pltpu.make_async_copy(bufs[b], acc.at[dst_v.at[t]], ssems[b]).wait()

        for b in range(NBUF):
            gather(b, b)

        @pl.loop(0, KFULL, step=NBUF)
        def _(j):
            for b in range(NBUF):
                gather_wait(j + b, b)
                scat(j + b, b)
            for b in range(NBUF):
                scat_wait(j + b, b)

                @pl.when(j + NBUF + b < K)
                def _():
                    gather(j + NBUF + b, b)

        for t in range(KFULL, K):
            b = t % NBUF
            gather_wait(t, b)
            pltpu.sync_copy(bufs[b], acc.at[dst_v.at[t]], add=True)

        plsc.subcore_barrier()
        pltpu.sync_copy(acc.at[pl.ds(s * RPT, RPT)],
                        out_hbm.at[c, pl.ds(s * RPT, RPT)])

    return k(table, src_slab, dst_slab, zrows)


def _sc_degree(dst_slab, zdeg):
    mesh = plsc.VectorSubcoreMesh(core_axis_name="c", subcore_axis_name="s")

    @functools.partial(
        pl.kernel,
        out_type=jax.ShapeDtypeStruct((NC, ACC_ROWS, DEGW), jnp.float32),
        mesh=mesh,
        scratch_types=[
            pltpu.VMEM((K, CHUNK), jnp.int32),
            pltpu.VMEM((CHUNK, DEGW), jnp.float32),
            pltpu.VMEM_SHARED((ACC_ROWS, DEGW), jnp.float32),
            pltpu.SemaphoreType.DMA,
        ],
        compiler_params=pltpu.CompilerParams(use_tc_tiling_on_sc=False),
    )
    def k(dst_hbm, z_hbm, out_hbm, dst_v, ones_v, acc, sem):
        c = lax.axis_index("c")
        s = lax.axis_index("s")
        pltpu.sync_copy(z_hbm, acc.at[pl.ds(s * RPT, RPT)])
        pltpu.sync_copy(dst_hbm.at[s], dst_v)
        for i in range(CHUNK):
            ones_v[i] = jnp.ones((DEGW,), jnp.float32)
        plsc.subcore_barrier()

        lo = c * KHALF
        hi = KHALF + c * (K - KHALF)

        @pl.loop(lo, hi)
        def _(j):
            pltpu.async_copy(ones_v, acc.at[dst_v.at[j]], sem, add=True)

        @pl.loop(lo, hi)
        def _(j):
            pltpu.make_async_copy(ones_v, acc.at[dst_v.at[j]], sem).wait()

        plsc.subcore_barrier()
        pltpu.sync_copy(acc.at[pl.ds(s * RPT, RPT)],
                        out_hbm.at[c, pl.ds(s * RPT, RPT)])

    return k(dst_slab, zdeg)


def _dis_block(deg_ref, br):
    return lax.rsqrt(deg_ref[0][:, 0:1] + deg_ref[1][:, 0:1] + 1.0)


def _tc1(x, W, accd):
    BR = 1000

    def body(x_ref, w_ref, deg_ref, y_ref):
        dis = _dis_block(deg_ref, BR)
        y = dis * jnp.dot(x_ref[...], w_ref[...],
                          preferred_element_type=jnp.float32)
        y_ref[0] = y[:, :DH]
        y_ref[1] = y[:, DH:]

    return pl.pallas_call(
        body,
        grid=(N // BR,),
        in_specs=[
            pl.BlockSpec((BR, D), lambda j: (j, 0)),
            pl.BlockSpec((D, D), lambda j: (0, 0)),
            pl.BlockSpec((NC, BR, DEGW), lambda j: (0, j, 0)),
        ],
        out_specs=pl.BlockSpec((NC, BR, DH), lambda j: (0, j, 0)),
        out_shape=jax.ShapeDtypeStruct((NC, N, DH), jnp.float32),
    )(x, W, accd)


def _tc2(acc, yt, accd, b, W):
    BR = 1000

    def body(a_ref, y_ref, deg_ref, b_ref, w_ref, o_ref):
        dis = _dis_block(deg_ref, BR)
        s = jnp.concatenate(
            [a_ref[0] + y_ref[0], a_ref[1] + y_ref[1]], axis=1)
        h = jnp.tanh(dis * s + b_ref[...])
        y2 = dis * jnp.dot(h, w_ref[...], preferred_element_type=jnp.float32)
        o_ref[0] = y2[:, :DH]
        o_ref[1] = y2[:, DH:]

    return pl.pallas_call(
        body,
        grid=(N // BR,),
        in_specs=[
            pl.BlockSpec((NC, BR, DH), lambda j: (0, j, 0)),
            pl.BlockSpec((NC, BR, DH), lambda j: (0, j, 0)),
            pl.BlockSpec((NC, BR, DEGW), lambda j: (0, j, 0)),
            pl.BlockSpec((1, D), lambda j: (0, 0)),
            pl.BlockSpec((D, D), lambda j: (0, 0)),
        ],
        out_specs=pl.BlockSpec((NC, BR, DH), lambda j: (0, j, 0)),
        out_shape=jax.ShapeDtypeStruct((NC, N, DH), jnp.float32),
    )(acc, yt, accd, b, W)


def _tc3(acc, yt, accd, b):
    BR = 1000

    def body(a_ref, y_ref, deg_ref, b_ref, o_ref):
        dis = _dis_block(deg_ref, BR)
        s = jnp.concatenate(
            [a_ref[0] + y_ref[0], a_ref[1] + y_ref[1]], axis=1)
        o_ref[...] = jnp.tanh(dis * s + b_ref[...])

    return pl.pallas_call(
        body,
        grid=(N // BR,),
        in_specs=[
            pl.BlockSpec((NC, BR, DH), lambda j: (0, j, 0)),
            pl.BlockSpec((NC, BR, DH), lambda j: (0, j, 0)),
            pl.BlockSpec((NC, BR, DEGW), lambda j: (0, j, 0)),
            pl.BlockSpec((1, D), lambda j: (0, 0)),
        ],
        out_specs=pl.BlockSpec((BR, D), lambda j: (j, 0)),
        out_shape=jax.ShapeDtypeStruct((N, D), jnp.float32),
    )(acc, yt, accd, b)


def kernel(x, edge_index, W1, b1, W2, b2):
    src = edge_index[0]
    dst = edge_index[1]
    pad = EPAD - E
    srcp = jnp.concatenate(
        [src, jnp.zeros((pad,), jnp.int32)]).reshape(NS, K, CHUNK)
    dstp = jnp.concatenate(
        [dst, jnp.full((pad,), N, jnp.int32)]).reshape(NS, K, CHUNK)
    zrows = jnp.zeros((RPT, DH), jnp.float32)

    accd = _sc_degree(dstp, jnp.zeros((RPT, DEGW), jnp.float32))

    y1t = _tc1(x, W1, accd)
    acc1 = _sc_scatter(y1t.reshape(NC * N, DH), srcp, dstp, zrows)
    y2t = _tc2(acc1, y1t, accd, b1.reshape(1, D), W2)
    acc2 = _sc_scatter(y2t.reshape(NC * N, DH), srcp, dstp, zrows)
    return _tc3(acc2, y2t, accd, b2.reshape(1, D))


# speedup vs baseline: 24.3714x; 1.0011x over previous
import functools

import jax
import jax.numpy as jnp
from jax import lax
from jax.experimental import pallas as pl
from jax.experimental.pallas import tpu as pltpu
from jax.experimental.pallas import tpu_sc as plsc

N = 10000
D = 128
DH = D // 2
NC, NS = 2, 16
LANES = 16
CHUNK = 128
E = 320000
K = -(-E // (NS * CHUNK))
EPAD = NS * K * CHUNK
ACC_ROWS = ((N + 1 + 127) // 128) * 128
RPT = ACC_ROWS // NS
NBUF = 6
KFULL = (K // NBUF) * NBUF
KHALF = -(-K // 2)
DEGW = 16


def _sc_scatter(table, src_slab, dst_slab, zrows):
    mesh = plsc.VectorSubcoreMesh(core_axis_name="c", subcore_axis_name="s")

    @functools.partial(
        pl.kernel,
        out_type=jax.ShapeDtypeStruct((NC, ACC_ROWS, DH), jnp.float32),
        mesh=mesh,
        scratch_types=[
            pltpu.VMEM((K, CHUNK), jnp.int32),
            pltpu.VMEM((K, CHUNK), jnp.int32),
            [pltpu.VMEM((CHUNK, DH), jnp.float32) for _ in range(NBUF)],
            pltpu.VMEM_SHARED((ACC_ROWS, DH), jnp.float32),
            [pltpu.SemaphoreType.DMA for _ in range(NBUF)],
            [pltpu.SemaphoreType.DMA for _ in range(NBUF)],
        ],
        compiler_params=pltpu.CompilerParams(use_tc_tiling_on_sc=False,
                                             skip_device_barrier=True),
    )
    def k(table_hbm, src_hbm, dst_hbm, z_hbm, out_hbm,
          src_v, dst_v, bufs, acc, gsems, ssems):
        c = lax.axis_index("c")
        s = lax.axis_index("s")
        pltpu.sync_copy(z_hbm, acc.at[pl.ds(s * RPT, RPT)])
        pltpu.sync_copy(src_hbm.at[s], src_v)
        pltpu.sync_copy(dst_hbm.at[s], dst_v)
        off = c * N

        @pl.loop(0, K)
        def _(r):
            for i in range(CHUNK // LANES):
                sl = pl.ds(i * LANES, LANES)
                src_v[r, sl] = src_v[r, sl] + off

        plsc.subcore_barrier()

        def gather(t, b):
            pltpu.async_copy(table_hbm.at[src_v.at[t]], bufs[b], gsems[b])

        def gather_wait(t, b):
            pltpu.make_async_copy(
                table_hbm.at[src_v.at[t]], bufs[b], gsems[b]).wait()

        def scat(t, b):
            pltpu.async_copy(bufs[b], acc.at[dst_v.at[t]], ssems[b], add=True)

        def scat_wait(t, b):
            pltpu.make_async_copy(bufs[b], acc.at[dst_v.at[t]], ssems[b]).wait()

        for b in range(NBUF):
            gather(b, b)

        @pl.loop(0, KFULL, step=NBUF)
        def _(j):
            for b in range(NBUF):
                gather_wait(j + b, b)
                scat(j + b, b)
            for b in range(NBUF):
                scat_wait(j + b, b)

                @pl.when(j + NBUF + b < K)
                def _():
                    gather(j + NBUF + b, b)

        for t in range(KFULL, K):
            b = t % NBUF
            gather_wait(t, b)
            pltpu.sync_copy(bufs[b], acc.at[dst_v.at[t]], add=True)

        plsc.subcore_barrier()
        pltpu.sync_copy(acc.at[pl.ds(s * RPT, RPT)],
                        out_hbm.at[c, pl.ds(s * RPT, RPT)])

    return k(table, src_slab, dst_slab, zrows)


def _sc_degree(dst_slab, zdeg):
    mesh = plsc.VectorSubcoreMesh(core_axis_name="c", subcore_axis_name="s")

    @functools.partial(
        pl.kernel,
        out_type=jax.ShapeDtypeStruct((NC, ACC_ROWS, DEGW), jnp.float32),
        mesh=mesh,
        scratch_types=[
            pltpu.VMEM((K, CHUNK), jnp.int32),
            pltpu.VMEM((CHUNK, DEGW), jnp.float32),
            pltpu.VMEM_SHARED((ACC_ROWS, DEGW), jnp.float32),
            pltpu.SemaphoreType.DMA,
        ],
        compiler_params=pltpu.CompilerParams(use_tc_tiling_on_sc=False,
                                             skip_device_barrier=True),
    )
    def k(dst_hbm, z_hbm, out_hbm, dst_v, ones_v, acc, sem):
        c = lax.axis_index("c")
        s = lax.axis_index("s")
        pltpu.sync_copy(z_hbm, acc.at[pl.ds(s * RPT, RPT)])
        pltpu.sync_copy(dst_hbm.at[s], dst_v)
        for i in range(CHUNK):
            ones_v[i] = jnp.ones((DEGW,), jnp.float32)
        plsc.subcore_barrier()

        lo = c * KHALF
        hi = KHALF + c * (K - KHALF)

        @pl.loop(lo, hi)
        def _(j):
            pltpu.async_copy(ones_v, acc.at[dst_v.at[j]], sem, add=True)

        @pl.loop(lo, hi)
        def _(j):
            pltpu.make_async_copy(ones_v, acc.at[dst_v.at[j]], sem).wait()

        plsc.subcore_barrier()
        pltpu.sync_copy(acc.at[pl.ds(s * RPT, RPT)],
                        out_hbm.at[c, pl.ds(s * RPT, RPT)])

    return k(dst_slab, zdeg)


def _dis_block(deg_ref, br):
    return lax.rsqrt(deg_ref[0][:, 0:1] + deg_ref[1][:, 0:1] + 1.0)


def _tc1(x, W, accd):
    BR = 1000

    def body(x_ref, w_ref, deg_ref, y_ref):
        dis = _dis_block(deg_ref, BR)
        y = dis * jnp.dot(x_ref[...], w_ref[...],
                          preferred_element_type=jnp.float32)
        y_ref[0] = y[:, :DH]
        y_ref[1] = y[:, DH:]

    return pl.pallas_call(
        body,
        grid=(N // BR,),
        in_specs=[
            pl.BlockSpec((BR, D), lambda j: (j, 0)),
            pl.BlockSpec((D, D), lambda j: (0, 0)),
            pl.BlockSpec((NC, BR, DEGW), lambda j: (0, j, 0)),
        ],
        out_specs=pl.BlockSpec((NC, BR, DH), lambda j: (0, j, 0)),
        out_shape=jax.ShapeDtypeStruct((NC, N, DH), jnp.float32),
        compiler_params=pltpu.CompilerParams(skip_device_barrier=True),
    )(x, W, accd)


def _tc2(acc, yt, accd, b, W):
    BR = 1000

    def body(a_ref, y_ref, deg_ref, b_ref, w_ref, o_ref):
        dis = _dis_block(deg_ref, BR)
        s = jnp.concatenate(
            [a_ref[0] + y_ref[0], a_ref[1] + y_ref[1]], axis=1)
        h = jnp.tanh(dis * s + b_ref[...])
        y2 = dis * jnp.dot(h, w_ref[...], preferred_element_type=jnp.float32)
        o_ref[0] = y2[:, :DH]
        o_ref[1] = y2[:, DH:]

    return pl.pallas_call(
        body,
        grid=(N // BR,),
        in_specs=[
            pl.BlockSpec((NC, BR, DH), lambda j: (0, j, 0)),
            pl.BlockSpec((NC, BR, DH), lambda j: (0, j, 0)),
            pl.BlockSpec((NC, BR, DEGW), lambda j: (0, j, 0)),
            pl.BlockSpec((1, D), lambda j: (0, 0)),
            pl.BlockSpec((D, D), lambda j: (0, 0)),
        ],
        out_specs=pl.BlockSpec((NC, BR, DH), lambda j: (0, j, 0)),
        out_shape=jax.ShapeDtypeStruct((NC, N, DH), jnp.float32),
        compiler_params=pltpu.CompilerParams(skip_device_barrier=True),
    )(acc, yt, accd, b, W)


def _tc3(acc, yt, accd, b):
    BR = 1000

    def body(a_ref, y_ref, deg_ref, b_ref, o_ref):
        dis = _dis_block(deg_ref, BR)
        s = jnp.concatenate(
            [a_ref[0] + y_ref[0], a_ref[1] + y_ref[1]], axis=1)
        o_ref[...] = jnp.tanh(dis * s + b_ref[...])

    return pl.pallas_call(
        body,
        grid=(N // BR,),
        in_specs=[
            pl.BlockSpec((NC, BR, DH), lambda j: (0, j, 0)),
            pl.BlockSpec((NC, BR, DH), lambda j: (0, j, 0)),
            pl.BlockSpec((NC, BR, DEGW), lambda j: (0, j, 0)),
            pl.BlockSpec((1, D), lambda j: (0, 0)),
        ],
        out_specs=pl.BlockSpec((BR, D), lambda j: (j, 0)),
        out_shape=jax.ShapeDtypeStruct((N, D), jnp.float32),
        compiler_params=pltpu.CompilerParams(skip_device_barrier=True),
    )(acc, yt, accd, b)


def kernel(x, edge_index, W1, b1, W2, b2):
    src = edge_index[0]
    dst = edge_index[1]
    pad = EPAD - E
    srcp = jnp.concatenate(
        [src, jnp.zeros((pad,), jnp.int32)]).reshape(NS, K, CHUNK)
    dstp = jnp.concatenate(
        [dst, jnp.full((pad,), N, jnp.int32)]).reshape(NS, K, CHUNK)
    zrows = jnp.zeros((RPT, DH), jnp.float32)

    accd = _sc_degree(dstp, jnp.zeros((RPT, DEGW), jnp.float32))

    y1t = _tc1(x, W1, accd)
    acc1 = _sc_scatter(y1t.reshape(NC * N, DH), srcp, dstp, zrows)
    y2t = _tc2(acc1, y1t, accd, b1.reshape(1, D), W2)
    acc2 = _sc_scatter(y2t.reshape(NC * N, DH), srcp, dstp, zrows)
    return _tc3(acc2, y2t, accd, b2.reshape(1, D))
